# baseline (speedup 1.0000x reference)
import functools

import jax
import jax.numpy as jnp
from jax import lax
from jax.experimental import pallas as pl
from jax.experimental.pallas import tpu as pltpu
from jax.experimental.pallas import tpu_sc as plsc

NC, NS = 2, 16
NW = NC * NS
UNIT = 12
NSAMP = 10
DF = 128
LANES = 16
HALF = UNIT * NSAMP
PROWS = 2 * UNIT
FRAC_C0 = 0.85
DG = DF // 2


@functools.partial(jax.jit, static_argnums=(1, 2, 3))
def _run(args, n0, n1, b_pad):
    features, idx = args
    u_max = max(n0, n1)
    mesh = plsc.VectorSubcoreMesh(core_axis_name="c", subcore_axis_name="s")

    @functools.partial(
        pl.kernel,
        mesh=mesh,
        compiler_params=pltpu.CompilerParams(use_tc_tiling_on_sc=False),
        out_type=jax.ShapeDtypeStruct((b_pad, DF), jnp.float32),
        scratch_types=[
            pltpu.VMEM((u_max, 2, HALF // 2), jnp.int32),
            pltpu.VMEM((2 * HALF, DG), jnp.float32),
            pltpu.VMEM((2 * HALF, DG), jnp.float32),
            pltpu.VMEM((PROWS, DF), jnp.float32),
            pltpu.VMEM((PROWS, DF), jnp.float32),
            pltpu.SemaphoreType.DMA,
            pltpu.SemaphoreType.DMA,
            pltpu.SemaphoreType.DMA,
            pltpu.SemaphoreType.DMA,
        ],
    )
    def k(feat_hbm, idx_hbm, out_hbm, idx_v, rows0, rows1, out0, out1,
          gsem0, gsem1, osem0, osem1):
        c = lax.axis_index("c")
        s = lax.axis_index("s")
        wid = s * NC + c
        pltpu.sync_copy(idx_hbm.at[wid], idx_v)
        cnt = jnp.where(c == 0, n0, n1)
        nsteps = cnt // 4
        npairs = cnt // 2
        out_base = (s * (n0 + n1) + c * n0) * UNIT

        def fire(p, rows, sem):
            for h in range(2):
                pltpu.async_copy(
                    feat_hbm.at[idx_v.at[2 * p + h, 0]],
                    rows.at[pl.ds(h * HALF, HALF // 2)], sem)
                pltpu.async_copy(
                    feat_hbm.at[idx_v.at[2 * p + h, 1]],
                    rows.at[pl.ds(h * HALF + HALF // 2, HALF // 2)], sem)

        def drain(p, rows, sem):
            for h in range(2):
                pltpu.make_async_copy(
                    feat_hbm.at[idx_v.at[2 * p + h, 0]],
                    rows.at[pl.ds(h * HALF, HALF // 2)], sem).wait()
                pltpu.make_async_copy(
                    feat_hbm.at[idx_v.at[2 * p + h, 1]],
                    rows.at[pl.ds(h * HALF + HALF // 2, HALF // 2)], sem).wait()

        def out_slice(p):
            return out_hbm.at[pl.ds(out_base + p * PROWS, PROWS)]

        def tree(v):
            while len(v) > 1:
                v = [a + b for a, b in zip(v[::2], v[1::2])] + (
                    [v[-1]] if len(v) % 2 else [])
            return v[0]

        def reduce(rows, out):
            def row(r, c2):
                base = r * NSAMP
                for g in range(DF // 32):
                    sl = pl.ds(g * LANES, LANES)
                    vi = [lax.bitcast_convert_type(rows[base + j, sl], jnp.int32)
                          for j in range(NSAMP)]
                    ev = [lax.bitcast_convert_type(v << 16, jnp.float32) for v in vi]
                    ov = [lax.bitcast_convert_type(v & jnp.int32(-65536), jnp.float32)
                          for v in vi]
                    out[r, pl.ds(32 * g, LANES)] = (
                        tree(ev) * jnp.float32(1.0 / NSAMP))
                    out[r, pl.ds(32 * g + LANES, LANES)] = (
                        tree(ov) * jnp.float32(1.0 / NSAMP))
                return c2

            lax.fori_loop(0, PROWS, row, 0)

        fire(0, rows0, gsem0)

        def step(t, carry):
            p0 = 2 * t
            p1 = 2 * t + 1
            fire(p1, rows1, gsem1)
            drain(p0, rows0, gsem0)

            @pl.when(t > 0)
            def _():
                pltpu.make_async_copy(out0, out_slice(p0), osem0).wait()

            reduce(rows0, out0)
            pltpu.async_copy(out0, out_slice(p0), osem0)

            @pl.when(t + 1 < nsteps)
            def _():
                fire(p0 + 2, rows0, gsem0)

            drain(p1, rows1, gsem1)

            @pl.when(t > 0)
            def _():
                pltpu.make_async_copy(out1, out_slice(p1), osem1).wait()

            reduce(rows1, out1)
            pltpu.async_copy(out1, out_slice(p1), osem1)
            return carry

        lax.fori_loop(0, nsteps, step, 0)
        pltpu.make_async_copy(out0, out_slice(npairs - 2), osem0).wait()
        pltpu.make_async_copy(out1, out_slice(npairs - 1), osem1).wait()

    return k(features, idx)


def _split(total_units):
    n0 = int(round(total_units * FRAC_C0 / 4.0)) * 4
    n0 = min(max(n0, 4), total_units - 4)
    return n0, total_units - n0


def _pack_table(features):
    n = features.shape[0]
    fb = features.astype(jnp.bfloat16).reshape(n, DF // 32, 2, LANES)
    fb = fb.transpose(0, 1, 3, 2)
    return jax.lax.bitcast_convert_type(fb, jnp.float32).reshape(n, DG)


def kernel(nodes, neighbors, features):
    del nodes
    b = neighbors.shape[0]
    u_total = -(-b // UNIT)
    per_s = -(-u_total // NS)
    per_s += (-per_s) % 8
    n0, n1 = _split(per_s)
    b_pad = NS * per_s * UNIT
    flat = neighbors.reshape(-1)
    pad = b_pad * NSAMP - flat.shape[0]
    if pad:
        flat = jnp.concatenate([flat, jnp.zeros((pad,), jnp.int32)])
    units = flat.reshape(NS, per_s, HALF)
    u_max = max(n0, n1)
    w0 = units[:, :n0, :]
    w1 = units[:, n0:, :]
    if n0 < u_max:
        w0 = jnp.pad(w0, ((0, 0), (0, u_max - n0), (0, 0)))
    if n1 < u_max:
        w1 = jnp.pad(w1, ((0, 0), (0, u_max - n1), (0, 0)))
    idx = jnp.stack([w0, w1], axis=1).reshape(NW, u_max, 2, HALF // 2)
    out = _run((_pack_table(features), idx), n0, n1, b_pad)
    return out[:b]


# speedup vs baseline: 1.0615x; 1.0615x over previous
import functools

import jax
import jax.numpy as jnp
from jax import lax
from jax.experimental import pallas as pl
from jax.experimental.pallas import tpu as pltpu
from jax.experimental.pallas import tpu_sc as plsc

NC, NS = 2, 16
NW = NC * NS
UNIT = 12
NSAMP = 10
DF = 128
LANES = 16
HALF = UNIT * NSAMP
PROWS = 2 * UNIT
FRAC_C0 = 0.85
DG = DF // 2


@functools.partial(jax.jit, static_argnums=(1, 2, 3))
def _run(args, n0, n1, b_pad):
    features, idx = args
    u_max = max(n0, n1)
    mesh = plsc.VectorSubcoreMesh(core_axis_name="c", subcore_axis_name="s")

    @functools.partial(
        pl.kernel,
        mesh=mesh,
        compiler_params=pltpu.CompilerParams(use_tc_tiling_on_sc=False),
        out_type=jax.ShapeDtypeStruct((b_pad, DF), jnp.float32),
        scratch_types=[
            pltpu.VMEM((u_max, HALF), jnp.int32),
            pltpu.VMEM((2 * HALF, DG), jnp.float32),
            pltpu.VMEM((2 * HALF, DG), jnp.float32),
            pltpu.VMEM((PROWS, DF), jnp.float32),
            pltpu.VMEM((PROWS, DF), jnp.float32),
            pltpu.SemaphoreType.DMA,
            pltpu.SemaphoreType.DMA,
            pltpu.SemaphoreType.DMA,
            pltpu.SemaphoreType.DMA,
        ],
    )
    def k(feat_hbm, idx_hbm, out_hbm, idx_v, rows0, rows1, out0, out1,
          gsem0, gsem1, osem0, osem1):
        c = lax.axis_index("c")
        s = lax.axis_index("s")
        wid = s * NC + c
        pltpu.sync_copy(idx_hbm.at[wid], idx_v)
        cnt = jnp.where(c == 0, n0, n1)
        nsteps = cnt // 4
        npairs = cnt // 2
        out_base = (s * (n0 + n1) + c * n0) * UNIT

        def fire(p, rows, sem):
            for h in range(2):
                pltpu.async_copy(
                    feat_hbm.at[idx_v.at[2 * p + h]],
                    rows.at[pl.ds(h * HALF, HALF)], sem)

        def drain(p, rows, sem):
            for h in range(2):
                pltpu.make_async_copy(
                    feat_hbm.at[idx_v.at[2 * p + h]],
                    rows.at[pl.ds(h * HALF, HALF)], sem).wait()

        def out_slice(p):
            return out_hbm.at[pl.ds(out_base + p * PROWS, PROWS)]

        def tree(v):
            while len(v) > 1:
                v = [a + b for a, b in zip(v[::2], v[1::2])] + (
                    [v[-1]] if len(v) % 2 else [])
            return v[0]

        def reduce(rows, out):
            def row(r, c2):
                base = r * NSAMP
                for g in range(DF // 32):
                    sl = pl.ds(g * LANES, LANES)
                    vi = [lax.bitcast_convert_type(rows[base + j, sl], jnp.int32)
                          for j in range(NSAMP)]
                    ev = [lax.bitcast_convert_type(v << 16, jnp.float32) for v in vi]
                    ov = [lax.bitcast_convert_type(v & jnp.int32(-65536), jnp.float32)
                          for v in vi]
                    out[r, pl.ds(32 * g, LANES)] = (
                        tree(ev) * jnp.float32(1.0 / NSAMP))
                    out[r, pl.ds(32 * g + LANES, LANES)] = (
                        tree(ov) * jnp.float32(1.0 / NSAMP))
                return c2

            lax.fori_loop(0, PROWS, row, 0)

        fire(0, rows0, gsem0)

        def step(t, carry):
            p0 = 2 * t
            p1 = 2 * t + 1
            fire(p1, rows1, gsem1)
            drain(p0, rows0, gsem0)

            @pl.when(t > 0)
            def _():
                pltpu.make_async_copy(out0, out_slice(p0), osem0).wait()

            reduce(rows0, out0)
            pltpu.async_copy(out0, out_slice(p0), osem0)

            @pl.when(t + 1 < nsteps)
            def _():
                fire(p0 + 2, rows0, gsem0)

            drain(p1, rows1, gsem1)

            @pl.when(t > 0)
            def _():
                pltpu.make_async_copy(out1, out_slice(p1), osem1).wait()

            reduce(rows1, out1)
            pltpu.async_copy(out1, out_slice(p1), osem1)
            return carry

        lax.fori_loop(0, nsteps, step, 0)
        pltpu.make_async_copy(out0, out_slice(npairs - 2), osem0).wait()
        pltpu.make_async_copy(out1, out_slice(npairs - 1), osem1).wait()

    return k(features, idx)


def _split(total_units):
    n0 = int(round(total_units * FRAC_C0 / 4.0)) * 4
    n0 = min(max(n0, 4), total_units - 4)
    return n0, total_units - n0


def _pack_table(features):
    n = features.shape[0]
    fb = features.astype(jnp.bfloat16).reshape(n, DF // 32, 2, LANES)
    fb = fb.transpose(0, 1, 3, 2)
    return jax.lax.bitcast_convert_type(fb, jnp.float32).reshape(n, DG)


def kernel(nodes, neighbors, features):
    del nodes
    b = neighbors.shape[0]
    u_total = -(-b // UNIT)
    per_s = -(-u_total // NS)
    per_s += (-per_s) % 8
    n0, n1 = _split(per_s)
    b_pad = NS * per_s * UNIT
    flat = neighbors.reshape(-1)
    pad = b_pad * NSAMP - flat.shape[0]
    if pad:
        flat = jnp.concatenate([flat, jnp.zeros((pad,), jnp.int32)])
    units = flat.reshape(NS, per_s, HALF)
    u_max = max(n0, n1)
    w0 = units[:, :n0, :]
    w1 = units[:, n0:, :]
    if n0 < u_max:
        w0 = jnp.pad(w0, ((0, 0), (0, u_max - n0), (0, 0)))
    if n1 < u_max:
        w1 = jnp.pad(w1, ((0, 0), (0, u_max - n1), (0, 0)))
    idx = jnp.stack([w0, w1], axis=1).reshape(NW, u_max, HALF)
    out = _run((_pack_table(features), idx), n0, n1, b_pad)
    return out[:b]


# speedup vs baseline: 1.0840x; 1.0212x over previous
import functools

import jax
import jax.numpy as jnp
from jax import lax
from jax.experimental import pallas as pl
from jax.experimental.pallas import tpu as pltpu
from jax.experimental.pallas import tpu_sc as plsc

NC, NS = 2, 16
NW = NC * NS
UNIT = 12
NSAMP = 10
DF = 128
LANES = 16
HALF = UNIT * NSAMP
PROWS = 2 * UNIT
FRAC_C0 = 0.85


@functools.partial(jax.jit, static_argnums=(1, 2, 3))
def _run(args, n0, n1, b_pad):
    features, idx = args
    u_max = max(n0, n1)
    per_s = n0 + n1
    mesh = plsc.VectorSubcoreMesh(core_axis_name="c", subcore_axis_name="s")

    @functools.partial(
        pl.kernel,
        mesh=mesh,
        compiler_params=pltpu.CompilerParams(use_tc_tiling_on_sc=False),
        out_type=jax.ShapeDtypeStruct((b_pad, DF), jnp.float32),
        scratch_types=[
            pltpu.VMEM((u_max, HALF), jnp.int32),
            pltpu.VMEM((2 * HALF, DF), jnp.float32),
            pltpu.VMEM((2 * HALF, DF), jnp.float32),
            pltpu.VMEM((PROWS, DF), jnp.float32),
            pltpu.VMEM((PROWS, DF), jnp.float32),
            pltpu.SemaphoreType.DMA,
            pltpu.SemaphoreType.DMA,
            pltpu.SemaphoreType.DMA,
            pltpu.SemaphoreType.DMA,
        ],
    )
    def k(feat_hbm, idx_hbm, out_hbm, idx_v, rows0, rows1, out0, out1,
          gsem0, gsem1, osem0, osem1):
        c = lax.axis_index("c")
        s = lax.axis_index("s")
        g0 = s * per_s + c * n0
        pltpu.sync_copy(idx_hbm.at[pl.ds(g0, u_max)], idx_v)
        cnt = jnp.where(c == 0, n0, n1)
        nsteps = cnt // 4
        npairs = cnt // 2
        out_base = g0 * UNIT

        def fire(p, rows, sem):
            for h in range(2):
                pltpu.async_copy(
                    feat_hbm.at[idx_v.at[2 * p + h]],
                    rows.at[pl.ds(h * HALF, HALF)], sem)

        def drain(p, rows, sem):
            for h in range(2):
                pltpu.make_async_copy(
                    feat_hbm.at[idx_v.at[2 * p + h]],
                    rows.at[pl.ds(h * HALF, HALF)], sem).wait()

        def out_slice(p):
            return out_hbm.at[pl.ds(out_base + p * PROWS, PROWS)]

        def tree(v):
            while len(v) > 1:
                v = [a + b for a, b in zip(v[::2], v[1::2])] + (
                    [v[-1]] if len(v) % 2 else [])
            return v[0]

        def reduce(rows, out):
            def row(r, c2):
                base = r * NSAMP
                for d in range(DF // LANES):
                    sl = pl.ds(d * LANES, LANES)
                    out[r, sl] = tree(
                        [rows[base + j, sl] for j in range(NSAMP)]
                    ) * jnp.float32(1.0 / NSAMP)
                return c2

            lax.fori_loop(0, PROWS, row, 0)

        fire(0, rows0, gsem0)

        def step(t, carry):
            p0 = 2 * t
            p1 = 2 * t + 1
            fire(p1, rows1, gsem1)
            drain(p0, rows0, gsem0)

            @pl.when(t > 0)
            def _():
                pltpu.make_async_copy(out0, out_slice(p0), osem0).wait()

            reduce(rows0, out0)
            pltpu.async_copy(out0, out_slice(p0), osem0)

            @pl.when(t + 1 < nsteps)
            def _():
                fire(p0 + 2, rows0, gsem0)

            drain(p1, rows1, gsem1)

            @pl.when(t > 0)
            def _():
                pltpu.make_async_copy(out1, out_slice(p1), osem1).wait()

            reduce(rows1, out1)
            pltpu.async_copy(out1, out_slice(p1), osem1)
            return carry

        lax.fori_loop(0, nsteps, step, 0)
        pltpu.make_async_copy(out0, out_slice(npairs - 2), osem0).wait()
        pltpu.make_async_copy(out1, out_slice(npairs - 1), osem1).wait()

    return k(features, idx)


def _split(total_units):
    n0 = int(round(total_units * FRAC_C0 / 4.0)) * 4
    n0 = min(max(n0, 4), total_units - 4)
    return n0, total_units - n0


def kernel(nodes, neighbors, features):
    del nodes
    b = neighbors.shape[0]
    u_total = -(-b // UNIT)
    per_s = -(-u_total // NS)
    per_s += (-per_s) % 8
    n0, n1 = _split(per_s)
    u_max = max(n0, n1)
    b_pad = NS * per_s * UNIT
    flat = neighbors.reshape(-1)
    pad = (NS * per_s + u_max) * HALF - flat.shape[0]
    flat = jnp.concatenate([flat, jnp.zeros((pad,), jnp.int32)])
    idx = flat.reshape(-1, HALF)
    out = _run((features, idx), n0, n1, b_pad)
    return out[:b]
